# baseline (speedup 1.0000x reference)
import functools

import jax
import jax.numpy as jnp
from jax import lax
from jax.experimental import pallas as pl
from jax.experimental.pallas import tpu as pltpu
from jax.experimental.pallas import tpu_sc as plsc

F32 = jnp.float32
BF16 = jnp.bfloat16
I32 = jnp.int32

NC = 2
NS = 16
NT = NC * NS

N_P = 10000
E_P = 320000
N_L = 4096
E_L = 65536
BATCH = 64
D = 128
NR = 6
CUT = 5.0
NBLK = 4

PE_T = E_P // NT
LE_T = E_L // NT
ZV_T = N_L // NT
PK = 80
PCH = PE_T // PK
LK = 64
LCH = LE_T // LK
N_PACC = 10240
PROWS_T = N_PACC // NS
LROWS_T = N_L // NS
ZR = 16
SB = 2000
NSUP = PE_T // SB
SCH = SB // PK
NV = D // 16

_sc_mesh = plsc.VectorSubcoreMesh(core_axis_name="c", subcore_axis_name="s")


def _leaky(x):
    return jnp.maximum(x, 0.01 * x)


@functools.partial(
    pl.kernel,
    out_type=(
        jax.ShapeDtypeStruct((E_P,), F32),
        jax.ShapeDtypeStruct((E_L,), F32),
        jax.ShapeDtypeStruct((N_L, D), F32),
    ),
    mesh=_sc_mesh,
    compiler_params=pltpu.CompilerParams(needs_layout_passes=False),
    scratch_types=[
        pltpu.VMEM((N_P,), F32),
        pltpu.VMEM((N_P,), F32),
        pltpu.VMEM((N_P,), F32),
        pltpu.VMEM((N_L,), F32),
        pltpu.VMEM((N_L,), F32),
        pltpu.VMEM((N_L,), F32),
        pltpu.VMEM((PE_T,), I32),
        pltpu.VMEM((PE_T,), I32),
        pltpu.VMEM((LE_T,), I32),
        pltpu.VMEM((LE_T,), I32),
        pltpu.VMEM((ZV_T,), I32),
        pltpu.VMEM((ZV_T, D), F32),
        pltpu.VMEM((PE_T,), F32),
        pltpu.VMEM((LE_T,), F32),
        pltpu.SemaphoreType.DMA,
    ],
)
def _sc_geom(ppx, ppy, ppz, psrc, pdst, lpx, lpy, lpz, lsrc, ldst, zidx, emb,
             psq_out, lsq_out, hl0_out,
             vpx, vpy, vpz, vlx, vly, vlz, vps, vpd, vls, vld, vz, vrows,
             vpsq, vlsq, sem):
    cid = lax.axis_index("c")
    sid = lax.axis_index("s")
    wid = cid * NS + sid

    pltpu.sync_copy(ppx, vpx)
    pltpu.sync_copy(ppy, vpy)
    pltpu.sync_copy(ppz, vpz)
    pltpu.sync_copy(lpx, vlx)
    pltpu.sync_copy(lpy, vly)
    pltpu.sync_copy(lpz, vlz)
    pltpu.sync_copy(psrc.at[pl.ds(wid * PE_T, PE_T)], vps)
    pltpu.sync_copy(pdst.at[pl.ds(wid * PE_T, PE_T)], vpd)
    pltpu.sync_copy(lsrc.at[pl.ds(wid * LE_T, LE_T)], vls)
    pltpu.sync_copy(ldst.at[pl.ds(wid * LE_T, LE_T)], vld)

    @pl.loop(0, PE_T // 16)
    def _(i):
        si = vps[pl.ds(i * 16, 16)]
        di = vpd[pl.ds(i * 16, 16)]
        dx = plsc.load_gather(vpx, [di]) - plsc.load_gather(vpx, [si])
        dy = plsc.load_gather(vpy, [di]) - plsc.load_gather(vpy, [si])
        dz = plsc.load_gather(vpz, [di]) - plsc.load_gather(vpz, [si])
        sq = dx * dx + dy * dy + dz * dz + 1e-9
        y = plsc.bitcast(0x5F3759DF - (plsc.bitcast(sq, I32) >> 1), F32)
        y = y * (1.5 - 0.5 * sq * y * y)
        y = y * (1.5 - 0.5 * sq * y * y)
        vpsq[pl.ds(i * 16, 16)] = jnp.exp(-(sq * y))

    @pl.loop(0, LE_T // 16)
    def _(i):
        si = vls[pl.ds(i * 16, 16)]
        di = vld[pl.ds(i * 16, 16)]
        dx = plsc.load_gather(vlx, [di]) - plsc.load_gather(vlx, [si])
        dy = plsc.load_gather(vly, [di]) - plsc.load_gather(vly, [si])
        dz = plsc.load_gather(vlz, [di]) - plsc.load_gather(vlz, [si])
        vlsq[pl.ds(i * 16, 16)] = dx * dx + dy * dy + dz * dz

    pltpu.sync_copy(vpsq, psq_out.at[pl.ds(wid * PE_T, PE_T)])
    pltpu.sync_copy(vlsq, lsq_out.at[pl.ds(wid * LE_T, LE_T)])

    pltpu.sync_copy(zidx.at[pl.ds(wid * ZV_T, ZV_T)], vz)
    pltpu.async_copy(emb.at[vz], vrows, sem).wait()
    pltpu.sync_copy(vrows, hl0_out.at[pl.ds(wid * ZV_T, ZV_T)])


@functools.partial(
    pl.kernel,
    out_type=jax.ShapeDtypeStruct((NC, N_PACC, D), F32),
    mesh=_sc_mesh,
    compiler_params=pltpu.CompilerParams(needs_layout_passes=False),
    scratch_types=[
        pltpu.VMEM_SHARED((N_PACC, D), F32),
        pltpu.VMEM((SB,), I32),
        pltpu.VMEM((SCH, PK), I32),
        pltpu.VMEM((SB,), F32),
        pltpu.VMEM((3, PK, D), F32),
        pltpu.VMEM((ZR, D), F32),
        pltpu.SemaphoreType.DMA,
        pltpu.SemaphoreType.DMA,
        pltpu.SemaphoreType.DMA,
        pltpu.SemaphoreType.DMA,
        pltpu.SemaphoreType.DMA,
        pltpu.SemaphoreType.DMA,
    ],
)
def _sc_pmsg(h, pw, srcidx, dstidx4, out, acc, vsrc, vdst, vw,
             gbuf3, zbuf, g0, g1, g2, s0, s1, s2):
    cid = lax.axis_index("c")
    sid = lax.axis_index("s")
    wid = cid * NS + sid

    @pl.loop(0, ZR)
    def _(r):
        for v in range(NV):
            zbuf[r, pl.ds(v * 16, 16)] = jnp.zeros((16,), F32)

    row0 = sid * PROWS_T

    @pl.loop(0, PROWS_T // ZR)
    def _(t):
        pltpu.sync_copy(zbuf, acc.at[pl.ds(row0 + t * ZR, ZR)])

    plsc.subcore_barrier()

    gsems = (g0, g1, g2)
    ssems = (s0, s1, s2)

    def _issue(j, b):
        pltpu.async_copy(h.at[vsrc.at[pl.ds(j * PK, PK)]], gbuf3.at[b],
                         gsems[b])

    def _drain(b):
        pltpu.make_async_copy(gbuf3.at[b], acc.at[vdst.at[0]],
                              ssems[b]).wait()

    def _process(j, b):
        buf = gbuf3.at[b]
        pltpu.make_async_copy(h.at[vsrc.at[pl.ds(0, PK)]], buf,
                              gsems[b]).wait()

        @pl.loop(0, PK // 16)
        def _(e16):
            wv = vw[pl.ds(j * PK + e16 * 16, 16)]
            for u in range(16):
                w = wv[u]
                r = e16 * 16 + u
                for v in range(NV):
                    buf[r, pl.ds(v * 16, 16)] = buf[r, pl.ds(v * 16, 16)] * w

        pltpu.async_copy(buf, acc.at[vdst.at[j]], ssems[b], add=True)

    @pl.loop(0, NSUP)
    def _(sc):
        base = wid * PE_T + sc * SB
        pltpu.sync_copy(srcidx.at[pl.ds(base, SB)], vsrc)
        pltpu.sync_copy(pw.at[pl.ds(base, SB)], vw)
        pltpu.sync_copy(dstidx4.at[wid, sc], vdst)

        _issue(0, 0)
        _issue(1, 1)
        for j in range(SCH):
            _process(j, j % 3)
            if j + 2 < SCH:
                if j >= 1:
                    _drain((j + 2) % 3)
                _issue(j + 2, (j + 2) % 3)
        for j in range(SCH - 3, SCH):
            _drain(j % 3)

    plsc.subcore_barrier()
    pltpu.sync_copy(acc.at[pl.ds(sid * PROWS_T, PROWS_T)],
                    out.at[cid, pl.ds(sid * PROWS_T, PROWS_T)])


@functools.partial(
    pl.kernel,
    out_type=jax.ShapeDtypeStruct((NC, N_L, D), F32),
    mesh=_sc_mesh,
    compiler_params=pltpu.CompilerParams(needs_layout_passes=False),
    scratch_types=[
        pltpu.VMEM_SHARED((N_L, D), F32),
        pltpu.VMEM((LE_T,), I32),
        pltpu.VMEM((LCH, LK), I32),
        pltpu.VMEM((3, LK, D), F32),
        pltpu.VMEM((3, LK, D), F32),
        pltpu.VMEM((3, LK, D), BF16),
        pltpu.VMEM((ZR, D), F32),
        pltpu.SemaphoreType.DMA,
        pltpu.SemaphoreType.DMA,
        pltpu.SemaphoreType.DMA,
        pltpu.SemaphoreType.DMA,
        pltpu.SemaphoreType.DMA,
        pltpu.SemaphoreType.DMA,
    ],
)
def _sc_lmsg(a_nodes, b_nodes, redge, srcidx, dstidx3, out,
             acc, vsrc, vdst, ga3, gb3, rbuf3, zbuf, g0, g1, g2, s0, s1, s2):
    cid = lax.axis_index("c")
    sid = lax.axis_index("s")
    wid = cid * NS + sid

    @pl.loop(0, ZR)
    def _(r):
        for v in range(NV):
            zbuf[r, pl.ds(v * 16, 16)] = jnp.zeros((16,), F32)

    row0 = sid * LROWS_T

    @pl.loop(0, LROWS_T // ZR)
    def _(t):
        pltpu.sync_copy(zbuf, acc.at[pl.ds(row0 + t * ZR, ZR)])

    plsc.subcore_barrier()

    pltpu.sync_copy(srcidx.at[pl.ds(wid * LE_T, LE_T)], vsrc)
    pltpu.sync_copy(dstidx3.at[wid], vdst)

    gsems = (g0, g1, g2)
    ssems = (s0, s1, s2)

    def _issue(j, b):
        sem = gsems[b]
        pltpu.async_copy(
            a_nodes.at[vsrc.at[pl.ds(j * LK, LK)]], ga3.at[b], sem)
        pltpu.async_copy(b_nodes.at[vdst.at[j]], gb3.at[b], sem)
        pltpu.async_copy(
            redge.at[pl.ds(wid * LE_T + j * LK, LK)], rbuf3.at[b], sem)

    def _drain(b):
        pltpu.make_async_copy(ga3.at[b], acc.at[vdst.at[0]], ssems[b]).wait()

    def _process(j, b):
        for _ in range(2):
            pltpu.make_async_copy(
                a_nodes.at[vsrc.at[pl.ds(0, LK)]], ga3.at[b],
                gsems[b]).wait()
        pltpu.make_async_copy(
            redge.at[pl.ds(0, LK)], rbuf3.at[b], gsems[b]).wait()
        ga = ga3.at[b]
        gb = gb3.at[b]
        rbuf = rbuf3.at[b]

        @pl.loop(0, LK)
        def _(e):
            for v in range(D // 32):
                rv = rbuf[e, pl.ds(v * 32, 32)]
                q0, q1 = plsc.unpack(rv, format=plsc.PackFormat.INTERLEAVED)
                x0 = (ga[e, pl.ds(v * 32, 16)] + gb[e, pl.ds(v * 32, 16)]
                      + q0)
                x1 = (ga[e, pl.ds(v * 32 + 16, 16)]
                      + gb[e, pl.ds(v * 32 + 16, 16)] + q1)
                ga[e, pl.ds(v * 32, 16)] = jnp.maximum(x0, 0.01 * x0)
                ga[e, pl.ds(v * 32 + 16, 16)] = jnp.maximum(x1, 0.01 * x1)

        pltpu.async_copy(ga, acc.at[vdst.at[j]], ssems[b], add=True)

    _issue(0, 0)
    _issue(1, 1)
    for j in range(LCH):
        _process(j, j % 3)
        if j + 2 < LCH:
            if j >= 1:
                _drain((j + 2) % 3)
            _issue(j + 2, (j + 2) % 3)
    for j in range(LCH - 3, LCH):
        _drain(j % 3)

    plsc.subcore_barrier()
    pltpu.sync_copy(acc.at[pl.ds(sid * LROWS_T, LROWS_T)],
                    out.at[cid, pl.ds(sid * LROWS_T, LROWS_T)])


def _tc_coef_body(lsq_ref, o_ref):
    sq = lsq_ref[...] + 1e-9
    ld = jnp.sqrt(sq)
    env = jnp.clip(1.0 - ld * (1.0 / CUT), 0.0, 1.0) ** 5
    scale = env / ld
    for k in range(NR):
        fk = jnp.float32(jnp.pi * (k + 1) / CUT)
        o_ref[k] = jnp.sin(ld * fk) * scale


_tc_coef = pl.pallas_call(
    _tc_coef_body, out_shape=jax.ShapeDtypeStruct((NR, E_L // D, D), F32))


def _tc_rbf_body(c_ref, wr_ref, *o_refs):
    ct = c_ref[...]
    for b in range(NBLK):
        o_refs[b][...] = lax.dot_general(
            ct, wr_ref[b], (((0,), (0,)), ((), ())),
            preferred_element_type=F32).astype(BF16)


_RBF_BM = 2048
_tc_rbf = pl.pallas_call(
    _tc_rbf_body,
    grid=(E_L // _RBF_BM,),
    in_specs=[
        pl.BlockSpec((NR, _RBF_BM), lambda i: (0, i)),
        pl.BlockSpec((NBLK, NR, D), lambda i: (0, 0, 0)),
    ],
    out_specs=[pl.BlockSpec((_RBF_BM, D), lambda i: (i, 0))
               for _ in range(NBLK)],
    out_shape=[jax.ShapeDtypeStruct((E_L, D), BF16) for _ in range(NBLK)],
)


def _tc_pnode_body(part_ref, h_ref, wmsg_ref, wself_ref, o_ref):
    m = part_ref[0] + part_ref[1]
    x = (jnp.dot(m, wmsg_ref[...], preferred_element_type=F32)
         + jnp.dot(h_ref[...], wself_ref[...], preferred_element_type=F32))
    o_ref[...] = _leaky(x)


_PN_BM = 1000
_tc_pnode = pl.pallas_call(
    _tc_pnode_body,
    grid=(N_P // _PN_BM,),
    in_specs=[
        pl.BlockSpec((NC, _PN_BM, D), lambda i: (0, i, 0)),
        pl.BlockSpec((_PN_BM, D), lambda i: (i, 0)),
        pl.BlockSpec((D, D), lambda i: (0, 0)),
        pl.BlockSpec((D, D), lambda i: (0, 0)),
    ],
    out_specs=pl.BlockSpec((_PN_BM, D), lambda i: (i, 0)),
    out_shape=jax.ShapeDtypeStruct((N_P, D), F32),
)


def _tc_ab_body(h_ref, ws_ref, wd_ref, a_ref, b_ref):
    hl = h_ref[...]
    a_ref[...] = jnp.dot(hl, ws_ref[...], preferred_element_type=F32)
    b_ref[...] = jnp.dot(hl, wd_ref[...], preferred_element_type=F32)


_AB_BM = 1024
_tc_ab = pl.pallas_call(
    _tc_ab_body,
    grid=(N_L // _AB_BM,),
    in_specs=[
        pl.BlockSpec((_AB_BM, D), lambda i: (i, 0)),
        pl.BlockSpec((D, D), lambda i: (0, 0)),
        pl.BlockSpec((D, D), lambda i: (0, 0)),
    ],
    out_specs=[
        pl.BlockSpec((_AB_BM, D), lambda i: (i, 0)),
        pl.BlockSpec((_AB_BM, D), lambda i: (i, 0)),
    ],
    out_shape=[
        jax.ShapeDtypeStruct((N_L, D), F32),
        jax.ShapeDtypeStruct((N_L, D), F32),
    ],
)


def _tc_lupd_body(part_ref, hl_ref, wu_ref, o_ref):
    g = part_ref[0] + part_ref[1]
    x = jnp.dot(g, wu_ref[...], preferred_element_type=F32)
    o_ref[...] = _leaky(x) + hl_ref[...]


_tc_lupd = pl.pallas_call(
    _tc_lupd_body,
    grid=(N_L // _AB_BM,),
    in_specs=[
        pl.BlockSpec((NC, _AB_BM, D), lambda i: (0, i, 0)),
        pl.BlockSpec((_AB_BM, D), lambda i: (i, 0)),
        pl.BlockSpec((D, D), lambda i: (0, 0)),
    ],
    out_specs=pl.BlockSpec((_AB_BM, D), lambda i: (i, 0)),
    out_shape=jax.ShapeDtypeStruct((N_L, D), F32),
)


def _make_tc_pool(n, bm):
    def body(x_ref, bat_ref, o_ref, acc, cnt):
        i = pl.program_id(0)

        @pl.when(i == 0)
        def _():
            acc[...] = jnp.zeros_like(acc)
            cnt[...] = jnp.zeros_like(cnt)

        oh = (bat_ref[...] == lax.broadcasted_iota(I32, (1, BATCH), 1)
              ).astype(F32)
        x = x_ref[...]
        acc[...] += lax.dot_general(oh, x, (((0,), (0,)), ((), ())),
                                    preferred_element_type=F32)
        cnt[...] += lax.dot_general(oh, jnp.ones_like(x),
                                    (((0,), (0,)), ((), ())),
                                    preferred_element_type=F32)

        @pl.when(i == pl.num_programs(0) - 1)
        def _():
            o_ref[...] = acc[...] / jnp.maximum(cnt[...], 1.0)

    return pl.pallas_call(
        body,
        grid=(n // bm,),
        in_specs=[
            pl.BlockSpec((bm, D), lambda i: (i, 0)),
            pl.BlockSpec((bm, 1), lambda i: (i, 0)),
        ],
        out_specs=pl.BlockSpec((BATCH, D), lambda i: (0, 0)),
        out_shape=jax.ShapeDtypeStruct((BATCH, D), F32),
        scratch_shapes=[
            pltpu.VMEM((BATCH, D), F32),
            pltpu.VMEM((BATCH, D), F32),
        ],
    )


_tc_pool_p = _make_tc_pool(N_P, 2000)
_tc_pool_l = _make_tc_pool(N_L, 1024)


def _tc_fuse_body(pm, lm, prob, wph, bph, wlh, blh, w1p, w1l, w1r, b1,
                  w2, b2, w3, b3, o_ref):
    dot = functools.partial(jnp.dot, preferred_element_type=F32)
    pe = _leaky(dot(pm[...], wph[...]) + bph[...])
    le = _leaky(dot(lm[...], wlh[...]) + blh[...])
    x = _leaky(dot(pe, w1p[...]) + dot(le, w1l[...])
               + prob[...] * w1r[...] + b1[...])
    x = _leaky(dot(x, w2[...]) + b2[...])
    o_ref[...] = jax.nn.sigmoid(dot(x, w3[...]) + b3[...])


_tc_fuse = pl.pallas_call(
    _tc_fuse_body, out_shape=jax.ShapeDtypeStruct((BATCH, 1), F32))


def kernel(protein_x, protein_pos, protein_edge_index, protein_batch,
           ligand_z, ligand_pos, ligand_edge_index, ligand_batch,
           pocket_probability, Wp_msg, Wp_self, Wp_head, bp_head, emb_table,
           W_msg_b, W_upd_b, Wl_head, bl_head, W1, b1, W2, b2, W3, b3):
    psrc = protein_edge_index[0]
    pdst = protein_edge_index[1]
    lsrc = ligand_edge_index[0]
    ldst = ligand_edge_index[1]
    pdst4 = pdst.reshape(NT, NSUP, SCH, PK)
    ldst3 = ldst.reshape(NT, LCH, LK)

    pw, lsq, hl0 = _sc_geom(
        protein_pos[:, 0], protein_pos[:, 1], protein_pos[:, 2],
        psrc, pdst,
        ligand_pos[:, 0], ligand_pos[:, 1], ligand_pos[:, 2],
        lsrc, ldst, ligand_z, emb_table)

    coef = _tc_coef(lsq.reshape(E_L // D, D))
    rperm = [32 * (k // 32) + (k % 32) // 2 + 16 * (k % 2) for k in range(D)]
    redge = _tc_rbf(coef.reshape(NR, E_L), W_msg_b[:, 2 * D:, :][:, :, rperm])

    h = protein_x
    for l in range(2):
        part = _sc_pmsg(h, pw, psrc, pdst4)
        h = _tc_pnode(part, h, Wp_msg[l], Wp_self[l])
    pm = _tc_pool_p(h, protein_batch.reshape(N_P, 1))

    hl = hl0
    for blk in range(NBLK):
        a, b = _tc_ab(hl, W_msg_b[blk, :D, :], W_msg_b[blk, D:2 * D, :])
        part = _sc_lmsg(a, b, redge[blk], lsrc, ldst3)
        hl = _tc_lupd(part, hl, W_upd_b[blk])
    lm = _tc_pool_l(hl, ligand_batch.reshape(N_L, 1))

    return _tc_fuse(
        pm, lm, pocket_probability.reshape(BATCH, 1),
        Wp_head, bp_head.reshape(1, D), Wl_head, bl_head.reshape(1, D),
        W1[:D], W1[D:2 * D], W1[2 * D:].reshape(1, 64), b1.reshape(1, 64),
        W2, b2.reshape(1, 16), W3, b3.reshape(1, 1))


# speedup vs baseline: 1.1401x; 1.1401x over previous
import functools

import jax
import jax.numpy as jnp
from jax import lax
from jax.experimental import pallas as pl
from jax.experimental.pallas import tpu as pltpu
from jax.experimental.pallas import tpu_sc as plsc

F32 = jnp.float32
I32 = jnp.int32

NC = 2
NS = 16
NT = NC * NS

N_P = 10000
E_P = 320000
N_L = 4096
E_L = 65536
BATCH = 64
D = 128
NR = 6
CUT = 5.0
NBLK = 4

PE_T = E_P // NT
LE_T = E_L // NT
ZV_T = N_L // NT
PK = 80
PCH = PE_T // PK
LK = 64
LCH = LE_T // LK
N_PACC = 10240
PROWS_T = N_PACC // NS
LROWS_T = N_L // NS
ZR = 16
SB = 2000
NSUP = PE_T // SB
SCH = SB // PK
NV = D // 16

_sc_mesh = plsc.VectorSubcoreMesh(core_axis_name="c", subcore_axis_name="s")


def _leaky(x):
    return jnp.maximum(x, 0.01 * x)


@functools.partial(
    pl.kernel,
    out_type=(
        jax.ShapeDtypeStruct((E_P,), F32),
        jax.ShapeDtypeStruct((E_L,), F32),
        jax.ShapeDtypeStruct((N_L, D), F32),
    ),
    mesh=_sc_mesh,
    compiler_params=pltpu.CompilerParams(needs_layout_passes=False),
    scratch_types=[
        pltpu.VMEM((N_P,), F32),
        pltpu.VMEM((N_P,), F32),
        pltpu.VMEM((N_P,), F32),
        pltpu.VMEM((N_L,), F32),
        pltpu.VMEM((N_L,), F32),
        pltpu.VMEM((N_L,), F32),
        pltpu.VMEM((PE_T,), I32),
        pltpu.VMEM((PE_T,), I32),
        pltpu.VMEM((LE_T,), I32),
        pltpu.VMEM((LE_T,), I32),
        pltpu.VMEM((ZV_T,), I32),
        pltpu.VMEM((ZV_T, D), F32),
        pltpu.VMEM((PE_T,), F32),
        pltpu.VMEM((LE_T,), F32),
        pltpu.SemaphoreType.DMA,
    ],
)
def _sc_geom(ppx, ppy, ppz, psrc, pdst, lpx, lpy, lpz, lsrc, ldst, zidx, emb,
             psq_out, lsq_out, hl0_out,
             vpx, vpy, vpz, vlx, vly, vlz, vps, vpd, vls, vld, vz, vrows,
             vpsq, vlsq, sem):
    cid = lax.axis_index("c")
    sid = lax.axis_index("s")
    wid = cid * NS + sid

    pltpu.sync_copy(ppx, vpx)
    pltpu.sync_copy(ppy, vpy)
    pltpu.sync_copy(ppz, vpz)
    pltpu.sync_copy(lpx, vlx)
    pltpu.sync_copy(lpy, vly)
    pltpu.sync_copy(lpz, vlz)
    pltpu.sync_copy(psrc.at[pl.ds(wid * PE_T, PE_T)], vps)
    pltpu.sync_copy(pdst.at[pl.ds(wid * PE_T, PE_T)], vpd)
    pltpu.sync_copy(lsrc.at[pl.ds(wid * LE_T, LE_T)], vls)
    pltpu.sync_copy(ldst.at[pl.ds(wid * LE_T, LE_T)], vld)

    @pl.loop(0, PE_T // 16)
    def _(i):
        si = vps[pl.ds(i * 16, 16)]
        di = vpd[pl.ds(i * 16, 16)]
        dx = plsc.load_gather(vpx, [di]) - plsc.load_gather(vpx, [si])
        dy = plsc.load_gather(vpy, [di]) - plsc.load_gather(vpy, [si])
        dz = plsc.load_gather(vpz, [di]) - plsc.load_gather(vpz, [si])
        sq = dx * dx + dy * dy + dz * dz + 1e-9
        y = plsc.bitcast(0x5F3759DF - (plsc.bitcast(sq, I32) >> 1), F32)
        y = y * (1.5 - 0.5 * sq * y * y)
        y = y * (1.5 - 0.5 * sq * y * y)
        vpsq[pl.ds(i * 16, 16)] = jnp.exp(-(sq * y))

    @pl.loop(0, LE_T // 16)
    def _(i):
        si = vls[pl.ds(i * 16, 16)]
        di = vld[pl.ds(i * 16, 16)]
        dx = plsc.load_gather(vlx, [di]) - plsc.load_gather(vlx, [si])
        dy = plsc.load_gather(vly, [di]) - plsc.load_gather(vly, [si])
        dz = plsc.load_gather(vlz, [di]) - plsc.load_gather(vlz, [si])
        vlsq[pl.ds(i * 16, 16)] = dx * dx + dy * dy + dz * dz

    pltpu.sync_copy(vpsq, psq_out.at[pl.ds(wid * PE_T, PE_T)])
    pltpu.sync_copy(vlsq, lsq_out.at[pl.ds(wid * LE_T, LE_T)])

    pltpu.sync_copy(zidx.at[pl.ds(wid * ZV_T, ZV_T)], vz)
    pltpu.async_copy(emb.at[vz], vrows, sem).wait()
    pltpu.sync_copy(vrows, hl0_out.at[pl.ds(wid * ZV_T, ZV_T)])


@functools.partial(
    pl.kernel,
    out_type=jax.ShapeDtypeStruct((NC, N_PACC, D), F32),
    mesh=_sc_mesh,
    compiler_params=pltpu.CompilerParams(needs_layout_passes=False),
    scratch_types=[
        pltpu.VMEM_SHARED((N_PACC, D), F32),
        pltpu.VMEM((SB,), I32),
        pltpu.VMEM((SCH, PK), I32),
        pltpu.VMEM((SB,), F32),
        pltpu.VMEM((3, PK, D), F32),
        pltpu.VMEM((ZR, D), F32),
        pltpu.SemaphoreType.DMA,
        pltpu.SemaphoreType.DMA,
        pltpu.SemaphoreType.DMA,
        pltpu.SemaphoreType.DMA,
        pltpu.SemaphoreType.DMA,
        pltpu.SemaphoreType.DMA,
    ],
)
def _sc_pmsg(h, pw, srcidx, dstidx4, out, acc, vsrc, vdst, vw,
             gbuf3, zbuf, g0, g1, g2, s0, s1, s2):
    cid = lax.axis_index("c")
    sid = lax.axis_index("s")
    wid = cid * NS + sid

    @pl.loop(0, ZR)
    def _(r):
        for v in range(NV):
            zbuf[r, pl.ds(v * 16, 16)] = jnp.zeros((16,), F32)

    row0 = sid * PROWS_T

    @pl.loop(0, PROWS_T // ZR)
    def _(t):
        pltpu.sync_copy(zbuf, acc.at[pl.ds(row0 + t * ZR, ZR)])

    plsc.subcore_barrier()

    gsems = (g0, g1, g2)
    ssems = (s0, s1, s2)

    def _issue(j, b):
        pltpu.async_copy(h.at[vsrc.at[pl.ds(j * PK, PK)]], gbuf3.at[b],
                         gsems[b])

    def _drain(b):
        pltpu.make_async_copy(gbuf3.at[b], acc.at[vdst.at[0]],
                              ssems[b]).wait()

    def _process(j, b):
        buf = gbuf3.at[b]
        pltpu.make_async_copy(h.at[vsrc.at[pl.ds(0, PK)]], buf,
                              gsems[b]).wait()

        @pl.loop(0, PK // 16)
        def _(e16):
            wv = vw[pl.ds(j * PK + e16 * 16, 16)]
            for u in range(16):
                w = wv[u]
                r = e16 * 16 + u
                for v in range(NV):
                    buf[r, pl.ds(v * 16, 16)] = buf[r, pl.ds(v * 16, 16)] * w

        pltpu.async_copy(buf, acc.at[vdst.at[j]], ssems[b], add=True)

    @pl.loop(0, NSUP)
    def _(sc):
        base = wid * PE_T + sc * SB
        pltpu.sync_copy(srcidx.at[pl.ds(base, SB)], vsrc)
        pltpu.sync_copy(pw.at[pl.ds(base, SB)], vw)
        pltpu.sync_copy(dstidx4.at[wid, sc], vdst)

        _issue(0, 0)
        _issue(1, 1)
        for j in range(SCH):
            _process(j, j % 3)
            if j + 2 < SCH:
                if j >= 1:
                    _drain((j + 2) % 3)
                _issue(j + 2, (j + 2) % 3)
        for j in range(SCH - 3, SCH):
            _drain(j % 3)

    plsc.subcore_barrier()
    pltpu.sync_copy(acc.at[pl.ds(sid * PROWS_T, PROWS_T)],
                    out.at[cid, pl.ds(sid * PROWS_T, PROWS_T)])


@functools.partial(
    pl.kernel,
    out_type=jax.ShapeDtypeStruct((NC, N_L, D), F32),
    mesh=_sc_mesh,
    compiler_params=pltpu.CompilerParams(needs_layout_passes=False),
    scratch_types=[
        pltpu.VMEM_SHARED((N_L, D), F32),
        pltpu.VMEM((LE_T,), I32),
        pltpu.VMEM((LCH, LK), I32),
        pltpu.VMEM((3, LK, D), F32),
        pltpu.VMEM((3, LK, D), F32),
        pltpu.VMEM((3, LK, D), F32),
        pltpu.VMEM((ZR, D), F32),
        pltpu.SemaphoreType.DMA,
        pltpu.SemaphoreType.DMA,
        pltpu.SemaphoreType.DMA,
        pltpu.SemaphoreType.DMA,
        pltpu.SemaphoreType.DMA,
        pltpu.SemaphoreType.DMA,
    ],
)
def _sc_lmsg(a_nodes, b_nodes, redge, srcidx, dstidx3, out,
             acc, vsrc, vdst, ga3, gb3, rbuf3, zbuf, g0, g1, g2, s0, s1, s2):
    cid = lax.axis_index("c")
    sid = lax.axis_index("s")
    wid = cid * NS + sid

    @pl.loop(0, ZR)
    def _(r):
        for v in range(NV):
            zbuf[r, pl.ds(v * 16, 16)] = jnp.zeros((16,), F32)

    row0 = sid * LROWS_T

    @pl.loop(0, LROWS_T // ZR)
    def _(t):
        pltpu.sync_copy(zbuf, acc.at[pl.ds(row0 + t * ZR, ZR)])

    plsc.subcore_barrier()

    pltpu.sync_copy(srcidx.at[pl.ds(wid * LE_T, LE_T)], vsrc)
    pltpu.sync_copy(dstidx3.at[wid], vdst)

    gsems = (g0, g1, g2)
    ssems = (s0, s1, s2)

    def _issue(j, b):
        sem = gsems[b]
        pltpu.async_copy(
            a_nodes.at[vsrc.at[pl.ds(j * LK, LK)]], ga3.at[b], sem)
        pltpu.async_copy(b_nodes.at[vdst.at[j]], gb3.at[b], sem)
        pltpu.async_copy(
            redge.at[pl.ds(wid * LE_T + j * LK, LK)], rbuf3.at[b], sem)

    def _drain(b):
        pltpu.make_async_copy(ga3.at[b], acc.at[vdst.at[0]], ssems[b]).wait()

    def _process(j, b):
        for _ in range(3):
            pltpu.make_async_copy(
                a_nodes.at[vsrc.at[pl.ds(0, LK)]], ga3.at[b],
                gsems[b]).wait()
        ga = ga3.at[b]
        gb = gb3.at[b]
        rbuf = rbuf3.at[b]

        @pl.loop(0, LK)
        def _(e):
            for v in range(NV):
                x = (ga[e, pl.ds(v * 16, 16)] + gb[e, pl.ds(v * 16, 16)]
                     + rbuf[e, pl.ds(v * 16, 16)])
                ga[e, pl.ds(v * 16, 16)] = jnp.maximum(x, 0.01 * x)

        pltpu.async_copy(ga, acc.at[vdst.at[j]], ssems[b], add=True)

    _issue(0, 0)
    _issue(1, 1)
    for j in range(LCH):
        _process(j, j % 3)
        if j + 2 < LCH:
            if j >= 1:
                _drain((j + 2) % 3)
            _issue(j + 2, (j + 2) % 3)
    for j in range(LCH - 3, LCH):
        _drain(j % 3)

    plsc.subcore_barrier()
    pltpu.sync_copy(acc.at[pl.ds(sid * LROWS_T, LROWS_T)],
                    out.at[cid, pl.ds(sid * LROWS_T, LROWS_T)])


def _tc_coef_body(lsq_ref, o_ref):
    sq = lsq_ref[...] + 1e-9
    ld = jnp.sqrt(sq)
    env = jnp.clip(1.0 - ld * (1.0 / CUT), 0.0, 1.0) ** 5
    scale = env / ld
    for k in range(NR):
        fk = jnp.float32(jnp.pi * (k + 1) / CUT)
        o_ref[k] = jnp.sin(ld * fk) * scale


_tc_coef = pl.pallas_call(
    _tc_coef_body, out_shape=jax.ShapeDtypeStruct((NR, E_L // D, D), F32))


def _tc_rbf_body(c_ref, wr_ref, *o_refs):
    ct = c_ref[...]
    for b in range(NBLK):
        o_refs[b][...] = lax.dot_general(
            ct, wr_ref[b], (((0,), (0,)), ((), ())),
            preferred_element_type=F32)


_RBF_BM = 2048
_tc_rbf = pl.pallas_call(
    _tc_rbf_body,
    grid=(E_L // _RBF_BM,),
    in_specs=[
        pl.BlockSpec((NR, _RBF_BM), lambda i: (0, i)),
        pl.BlockSpec((NBLK, NR, D), lambda i: (0, 0, 0)),
    ],
    out_specs=[pl.BlockSpec((_RBF_BM, D), lambda i: (i, 0))
               for _ in range(NBLK)],
    out_shape=[jax.ShapeDtypeStruct((E_L, D), F32) for _ in range(NBLK)],
)


def _tc_pnode_body(part_ref, h_ref, wmsg_ref, wself_ref, o_ref):
    m = part_ref[0] + part_ref[1]
    x = (jnp.dot(m, wmsg_ref[...], preferred_element_type=F32)
         + jnp.dot(h_ref[...], wself_ref[...], preferred_element_type=F32))
    o_ref[...] = _leaky(x)


_PN_BM = 1000
_tc_pnode = pl.pallas_call(
    _tc_pnode_body,
    grid=(N_P // _PN_BM,),
    in_specs=[
        pl.BlockSpec((NC, _PN_BM, D), lambda i: (0, i, 0)),
        pl.BlockSpec((_PN_BM, D), lambda i: (i, 0)),
        pl.BlockSpec((D, D), lambda i: (0, 0)),
        pl.BlockSpec((D, D), lambda i: (0, 0)),
    ],
    out_specs=pl.BlockSpec((_PN_BM, D), lambda i: (i, 0)),
    out_shape=jax.ShapeDtypeStruct((N_P, D), F32),
)


def _tc_ab_body(h_ref, ws_ref, wd_ref, a_ref, b_ref):
    hl = h_ref[...]
    a_ref[...] = jnp.dot(hl, ws_ref[...], preferred_element_type=F32)
    b_ref[...] = jnp.dot(hl, wd_ref[...], preferred_element_type=F32)


_AB_BM = 1024
_tc_ab = pl.pallas_call(
    _tc_ab_body,
    grid=(N_L // _AB_BM,),
    in_specs=[
        pl.BlockSpec((_AB_BM, D), lambda i: (i, 0)),
        pl.BlockSpec((D, D), lambda i: (0, 0)),
        pl.BlockSpec((D, D), lambda i: (0, 0)),
    ],
    out_specs=[
        pl.BlockSpec((_AB_BM, D), lambda i: (i, 0)),
        pl.BlockSpec((_AB_BM, D), lambda i: (i, 0)),
    ],
    out_shape=[
        jax.ShapeDtypeStruct((N_L, D), F32),
        jax.ShapeDtypeStruct((N_L, D), F32),
    ],
)


def _tc_lupd_body(part_ref, hl_ref, wu_ref, o_ref):
    g = part_ref[0] + part_ref[1]
    x = jnp.dot(g, wu_ref[...], preferred_element_type=F32)
    o_ref[...] = _leaky(x) + hl_ref[...]


_tc_lupd = pl.pallas_call(
    _tc_lupd_body,
    grid=(N_L // _AB_BM,),
    in_specs=[
        pl.BlockSpec((NC, _AB_BM, D), lambda i: (0, i, 0)),
        pl.BlockSpec((_AB_BM, D), lambda i: (i, 0)),
        pl.BlockSpec((D, D), lambda i: (0, 0)),
    ],
    out_specs=pl.BlockSpec((_AB_BM, D), lambda i: (i, 0)),
    out_shape=jax.ShapeDtypeStruct((N_L, D), F32),
)


def _tc_lupd_ab_body(part_ref, hl_ref, wu_ref, ws_ref, wd_ref,
                     o_ref, a_ref, b_ref):
    g = part_ref[0] + part_ref[1]
    x = jnp.dot(g, wu_ref[...], preferred_element_type=F32)
    hl2 = _leaky(x) + hl_ref[...]
    o_ref[...] = hl2
    a_ref[...] = jnp.dot(hl2, ws_ref[...], preferred_element_type=F32)
    b_ref[...] = jnp.dot(hl2, wd_ref[...], preferred_element_type=F32)


_tc_lupd_ab = pl.pallas_call(
    _tc_lupd_ab_body,
    grid=(N_L // _AB_BM,),
    in_specs=[
        pl.BlockSpec((NC, _AB_BM, D), lambda i: (0, i, 0)),
        pl.BlockSpec((_AB_BM, D), lambda i: (i, 0)),
        pl.BlockSpec((D, D), lambda i: (0, 0)),
        pl.BlockSpec((D, D), lambda i: (0, 0)),
        pl.BlockSpec((D, D), lambda i: (0, 0)),
    ],
    out_specs=[pl.BlockSpec((_AB_BM, D), lambda i: (i, 0))] * 3,
    out_shape=[jax.ShapeDtypeStruct((N_L, D), F32)] * 3,
)


def _make_tc_pool(n, bm):
    def body(x_ref, bat_ref, o_ref, acc, cnt):
        i = pl.program_id(0)

        @pl.when(i == 0)
        def _():
            acc[...] = jnp.zeros_like(acc)
            cnt[...] = jnp.zeros_like(cnt)

        oh = (bat_ref[...] == lax.broadcasted_iota(I32, (1, BATCH), 1)
              ).astype(F32)
        x = x_ref[...]
        acc[...] += lax.dot_general(oh, x, (((0,), (0,)), ((), ())),
                                    preferred_element_type=F32)
        cnt[...] += lax.dot_general(oh, jnp.ones_like(x),
                                    (((0,), (0,)), ((), ())),
                                    preferred_element_type=F32)

        @pl.when(i == pl.num_programs(0) - 1)
        def _():
            o_ref[...] = acc[...] / jnp.maximum(cnt[...], 1.0)

    return pl.pallas_call(
        body,
        grid=(n // bm,),
        in_specs=[
            pl.BlockSpec((bm, D), lambda i: (i, 0)),
            pl.BlockSpec((bm, 1), lambda i: (i, 0)),
        ],
        out_specs=pl.BlockSpec((BATCH, D), lambda i: (0, 0)),
        out_shape=jax.ShapeDtypeStruct((BATCH, D), F32),
        scratch_shapes=[
            pltpu.VMEM((BATCH, D), F32),
            pltpu.VMEM((BATCH, D), F32),
        ],
    )


_tc_pool_p = _make_tc_pool(N_P, 2000)
_tc_pool_l = _make_tc_pool(N_L, 1024)


def _tc_fuse_body(pm, lm, prob, wph, bph, wlh, blh, w1p, w1l, w1r, b1,
                  w2, b2, w3, b3, o_ref):
    dot = functools.partial(jnp.dot, preferred_element_type=F32)
    pe = _leaky(dot(pm[...], wph[...]) + bph[...])
    le = _leaky(dot(lm[...], wlh[...]) + blh[...])
    x = _leaky(dot(pe, w1p[...]) + dot(le, w1l[...])
               + prob[...] * w1r[...] + b1[...])
    x = _leaky(dot(x, w2[...]) + b2[...])
    o_ref[...] = jax.nn.sigmoid(dot(x, w3[...]) + b3[...])


_tc_fuse = pl.pallas_call(
    _tc_fuse_body, out_shape=jax.ShapeDtypeStruct((BATCH, 1), F32))


def kernel(protein_x, protein_pos, protein_edge_index, protein_batch,
           ligand_z, ligand_pos, ligand_edge_index, ligand_batch,
           pocket_probability, Wp_msg, Wp_self, Wp_head, bp_head, emb_table,
           W_msg_b, W_upd_b, Wl_head, bl_head, W1, b1, W2, b2, W3, b3):
    psrc = protein_edge_index[0]
    pdst = protein_edge_index[1]
    lsrc = ligand_edge_index[0]
    ldst = ligand_edge_index[1]
    pdst4 = pdst.reshape(NT, NSUP, SCH, PK)
    ldst3 = ldst.reshape(NT, LCH, LK)

    pw, lsq, hl0 = _sc_geom(
        protein_pos[:, 0], protein_pos[:, 1], protein_pos[:, 2],
        psrc, pdst,
        ligand_pos[:, 0], ligand_pos[:, 1], ligand_pos[:, 2],
        lsrc, ldst, ligand_z, emb_table)

    coef = _tc_coef(lsq.reshape(E_L // D, D))
    redge = _tc_rbf(coef.reshape(NR, E_L), W_msg_b[:, 2 * D:, :])

    h = protein_x
    for l in range(2):
        part = _sc_pmsg(h, pw, psrc, pdst4)
        h = _tc_pnode(part, h, Wp_msg[l], Wp_self[l])
    pm = _tc_pool_p(h, protein_batch.reshape(N_P, 1))

    hl = hl0
    a, b = _tc_ab(hl, W_msg_b[0, :D, :], W_msg_b[0, D:2 * D, :])
    for blk in range(NBLK):
        part = _sc_lmsg(a, b, redge[blk], lsrc, ldst3)
        if blk < NBLK - 1:
            hl, a, b = _tc_lupd_ab(
                part, hl, W_upd_b[blk],
                W_msg_b[blk + 1, :D, :], W_msg_b[blk + 1, D:2 * D, :])
        else:
            hl = _tc_lupd(part, hl, W_upd_b[blk])
    lm = _tc_pool_l(hl, ligand_batch.reshape(N_L, 1))

    return _tc_fuse(
        pm, lm, pocket_probability.reshape(BATCH, 1),
        Wp_head, bp_head.reshape(1, D), Wl_head, bl_head.reshape(1, D),
        W1[:D], W1[D:2 * D], W1[2 * D:].reshape(1, 64), b1.reshape(1, 64),
        W2, b2.reshape(1, 16), W3, b3.reshape(1, 1))


# speedup vs baseline: 1.1451x; 1.0044x over previous
import functools

import jax
import jax.numpy as jnp
from jax import lax
from jax.experimental import pallas as pl
from jax.experimental.pallas import tpu as pltpu
from jax.experimental.pallas import tpu_sc as plsc

F32 = jnp.float32
I32 = jnp.int32

NC = 2
NS = 16
NT = NC * NS

N_P = 10000
E_P = 320000
N_L = 4096
E_L = 65536
BATCH = 64
D = 128
NR = 6
CUT = 5.0
NBLK = 4

PE_T = E_P // NT
LE_T = E_L // NT
ZV_T = N_L // NT
PK = 80
PCH = PE_T // PK
LK = 64
LCH = LE_T // LK
N_PACC = 10240
PROWS_T = N_PACC // NS
LROWS_T = N_L // NS
ZR = 16
SB = 2000
NSUP = PE_T // SB
SCH = SB // PK
NV = D // 16

_sc_mesh = plsc.VectorSubcoreMesh(core_axis_name="c", subcore_axis_name="s")


def _leaky(x):
    return jnp.maximum(x, 0.01 * x)


@functools.partial(
    pl.kernel,
    out_type=(
        jax.ShapeDtypeStruct((E_P,), F32),
        jax.ShapeDtypeStruct((E_L,), F32),
        jax.ShapeDtypeStruct((N_L, D), F32),
    ),
    mesh=_sc_mesh,
    compiler_params=pltpu.CompilerParams(needs_layout_passes=False),
    scratch_types=[
        pltpu.VMEM((N_P,), F32),
        pltpu.VMEM((N_P,), F32),
        pltpu.VMEM((N_P,), F32),
        pltpu.VMEM((N_L,), F32),
        pltpu.VMEM((N_L,), F32),
        pltpu.VMEM((N_L,), F32),
        pltpu.VMEM((PE_T,), I32),
        pltpu.VMEM((PE_T,), I32),
        pltpu.VMEM((LE_T,), I32),
        pltpu.VMEM((LE_T,), I32),
        pltpu.VMEM((ZV_T,), I32),
        pltpu.VMEM((ZV_T, D), F32),
        pltpu.VMEM((PE_T,), F32),
        pltpu.VMEM((LE_T,), F32),
        pltpu.SemaphoreType.DMA,
    ],
)
def _sc_geom(ppx, ppy, ppz, psrc, pdst, lpx, lpy, lpz, lsrc, ldst, zidx, emb,
             psq_out, lsq_out, hl0_out,
             vpx, vpy, vpz, vlx, vly, vlz, vps, vpd, vls, vld, vz, vrows,
             vpsq, vlsq, sem):
    cid = lax.axis_index("c")
    sid = lax.axis_index("s")
    wid = cid * NS + sid

    pltpu.sync_copy(ppx, vpx)
    pltpu.sync_copy(ppy, vpy)
    pltpu.sync_copy(ppz, vpz)
    pltpu.sync_copy(lpx, vlx)
    pltpu.sync_copy(lpy, vly)
    pltpu.sync_copy(lpz, vlz)
    pltpu.sync_copy(psrc.at[pl.ds(wid * PE_T, PE_T)], vps)
    pltpu.sync_copy(pdst.at[pl.ds(wid * PE_T, PE_T)], vpd)
    pltpu.sync_copy(lsrc.at[pl.ds(wid * LE_T, LE_T)], vls)
    pltpu.sync_copy(ldst.at[pl.ds(wid * LE_T, LE_T)], vld)

    @pl.loop(0, PE_T // 16)
    def _(i):
        si = vps[pl.ds(i * 16, 16)]
        di = vpd[pl.ds(i * 16, 16)]
        dx = plsc.load_gather(vpx, [di]) - plsc.load_gather(vpx, [si])
        dy = plsc.load_gather(vpy, [di]) - plsc.load_gather(vpy, [si])
        dz = plsc.load_gather(vpz, [di]) - plsc.load_gather(vpz, [si])
        sq = dx * dx + dy * dy + dz * dz + 1e-9
        y = plsc.bitcast(0x5F3759DF - (plsc.bitcast(sq, I32) >> 1), F32)
        y = y * (1.5 - 0.5 * sq * y * y)
        y = y * (1.5 - 0.5 * sq * y * y)
        vpsq[pl.ds(i * 16, 16)] = jnp.exp(-(sq * y))

    @pl.loop(0, LE_T // 16)
    def _(i):
        si = vls[pl.ds(i * 16, 16)]
        di = vld[pl.ds(i * 16, 16)]
        dx = plsc.load_gather(vlx, [di]) - plsc.load_gather(vlx, [si])
        dy = plsc.load_gather(vly, [di]) - plsc.load_gather(vly, [si])
        dz = plsc.load_gather(vlz, [di]) - plsc.load_gather(vlz, [si])
        vlsq[pl.ds(i * 16, 16)] = dx * dx + dy * dy + dz * dz

    pltpu.sync_copy(vpsq, psq_out.at[pl.ds(wid * PE_T, PE_T)])
    pltpu.sync_copy(vlsq, lsq_out.at[pl.ds(wid * LE_T, LE_T)])

    pltpu.sync_copy(zidx.at[pl.ds(wid * ZV_T, ZV_T)], vz)
    pltpu.async_copy(emb.at[vz], vrows, sem).wait()
    pltpu.sync_copy(vrows, hl0_out.at[pl.ds(wid * ZV_T, ZV_T)])


@functools.partial(
    pl.kernel,
    out_type=jax.ShapeDtypeStruct((NC, N_PACC, D), F32),
    mesh=_sc_mesh,
    compiler_params=pltpu.CompilerParams(needs_layout_passes=False),
    scratch_types=[
        pltpu.VMEM_SHARED((N_PACC, D), F32),
        pltpu.VMEM((SB,), I32),
        pltpu.VMEM((SCH, PK), I32),
        pltpu.VMEM((SB,), F32),
        pltpu.VMEM((3, PK, D), F32),
        pltpu.VMEM((ZR, D), F32),
        pltpu.SemaphoreType.DMA,
        pltpu.SemaphoreType.DMA,
        pltpu.SemaphoreType.DMA,
        pltpu.SemaphoreType.DMA,
        pltpu.SemaphoreType.DMA,
        pltpu.SemaphoreType.DMA,
    ],
)
def _sc_pmsg(h, pw, srcidx, dstidx4, out, acc, vsrc, vdst, vw,
             gbuf3, zbuf, g0, g1, g2, s0, s1, s2):
    cid = lax.axis_index("c")
    sid = lax.axis_index("s")
    wid = cid * NS + sid

    @pl.loop(0, ZR)
    def _(r):
        for v in range(NV):
            zbuf[r, pl.ds(v * 16, 16)] = jnp.zeros((16,), F32)

    row0 = sid * PROWS_T

    @pl.loop(0, PROWS_T // ZR)
    def _(t):
        pltpu.sync_copy(zbuf, acc.at[pl.ds(row0 + t * ZR, ZR)])

    plsc.subcore_barrier()

    gsems = (g0, g1, g2)
    ssems = (s0, s1, s2)

    def _issue(j, b):
        pltpu.async_copy(h.at[vsrc.at[pl.ds(j * PK, PK)]], gbuf3.at[b],
                         gsems[b])

    def _drain(b):
        pltpu.make_async_copy(gbuf3.at[b], acc.at[vdst.at[0]],
                              ssems[b]).wait()

    def _process(j, b):
        buf = gbuf3.at[b]
        pltpu.make_async_copy(h.at[vsrc.at[pl.ds(0, PK)]], buf,
                              gsems[b]).wait()

        @pl.loop(0, PK // 16)
        def _(e16):
            wv = vw[pl.ds(j * PK + e16 * 16, 16)]
            for u in range(16):
                w = wv[u]
                r = e16 * 16 + u
                for v in range(NV):
                    buf[r, pl.ds(v * 16, 16)] = buf[r, pl.ds(v * 16, 16)] * w

        pltpu.async_copy(buf, acc.at[vdst.at[j]], ssems[b], add=True)

    @pl.loop(0, NSUP)
    def _(sc):
        base = wid * PE_T + sc * SB
        pltpu.sync_copy(srcidx.at[pl.ds(base, SB)], vsrc)
        pltpu.sync_copy(pw.at[pl.ds(base, SB)], vw)
        pltpu.sync_copy(dstidx4.at[wid, sc], vdst)

        _issue(0, 0)
        _issue(1, 1)
        for j in range(SCH):
            _process(j, j % 3)
            if j + 2 < SCH:
                if j >= 1:
                    _drain((j + 2) % 3)
                _issue(j + 2, (j + 2) % 3)
        for j in range(SCH - 3, SCH):
            _drain(j % 3)

    plsc.subcore_barrier()
    pltpu.sync_copy(acc.at[pl.ds(sid * PROWS_T, PROWS_T)],
                    out.at[cid, pl.ds(sid * PROWS_T, PROWS_T)])


@functools.partial(
    pl.kernel,
    out_type=jax.ShapeDtypeStruct((NC, N_L, D), F32),
    mesh=_sc_mesh,
    compiler_params=pltpu.CompilerParams(needs_layout_passes=False),
    scratch_types=[
        pltpu.VMEM_SHARED((N_L, D), F32),
        pltpu.VMEM((LE_T,), I32),
        pltpu.VMEM((LCH, LK), I32),
        pltpu.VMEM((3, LK, D), F32),
        pltpu.VMEM((3, LK, D), F32),
        pltpu.VMEM((3, LK, D), F32),
        pltpu.VMEM((ZR, D), F32),
        pltpu.SemaphoreType.DMA,
        pltpu.SemaphoreType.DMA,
        pltpu.SemaphoreType.DMA,
        pltpu.SemaphoreType.DMA,
        pltpu.SemaphoreType.DMA,
        pltpu.SemaphoreType.DMA,
    ],
)
def _sc_lmsg(a_nodes, b_nodes, redge, srcidx, dstidx3, out,
             acc, vsrc, vdst, ga3, gb3, rbuf3, zbuf, g0, g1, g2, s0, s1, s2):
    cid = lax.axis_index("c")
    sid = lax.axis_index("s")
    wid = cid * NS + sid

    @pl.loop(0, ZR)
    def _(r):
        for v in range(NV):
            zbuf[r, pl.ds(v * 16, 16)] = jnp.zeros((16,), F32)

    row0 = sid * LROWS_T

    @pl.loop(0, LROWS_T // ZR)
    def _(t):
        pltpu.sync_copy(zbuf, acc.at[pl.ds(row0 + t * ZR, ZR)])

    plsc.subcore_barrier()

    pltpu.sync_copy(srcidx.at[pl.ds(wid * LE_T, LE_T)], vsrc)
    pltpu.sync_copy(dstidx3.at[wid], vdst)

    gsems = (g0, g1, g2)
    ssems = (s0, s1, s2)

    def _issue(j, b):
        sem = gsems[b]
        pltpu.async_copy(
            a_nodes.at[vsrc.at[pl.ds(j * LK, LK)]], ga3.at[b], sem)
        pltpu.async_copy(b_nodes.at[vdst.at[j]], gb3.at[b], sem)
        pltpu.async_copy(
            redge.at[pl.ds(wid * LE_T + j * LK, LK)], rbuf3.at[b], sem)

    def _drain(b):
        pltpu.make_async_copy(ga3.at[b], acc.at[vdst.at[0]], ssems[b]).wait()

    def _process(j, b):
        for _ in range(3):
            pltpu.make_async_copy(
                a_nodes.at[vsrc.at[pl.ds(0, LK)]], ga3.at[b],
                gsems[b]).wait()
        ga = ga3.at[b]
        gb = gb3.at[b]
        rbuf = rbuf3.at[b]

        @pl.loop(0, LK)
        def _(e):
            for v in range(NV):
                x = (ga[e, pl.ds(v * 16, 16)] + gb[e, pl.ds(v * 16, 16)]
                     + rbuf[e, pl.ds(v * 16, 16)])
                ga[e, pl.ds(v * 16, 16)] = jnp.maximum(x, 0.01 * x)

        pltpu.async_copy(ga, acc.at[vdst.at[j]], ssems[b], add=True)

    _issue(0, 0)
    _issue(1, 1)
    for j in range(LCH):
        _process(j, j % 3)
        if j + 2 < LCH:
            if j >= 1:
                _drain((j + 2) % 3)
            _issue(j + 2, (j + 2) % 3)
    for j in range(LCH - 3, LCH):
        _drain(j % 3)

    plsc.subcore_barrier()
    pltpu.sync_copy(acc.at[pl.ds(sid * LROWS_T, LROWS_T)],
                    out.at[cid, pl.ds(sid * LROWS_T, LROWS_T)])


def _tc_coef_body(lsq_ref, o_ref):
    sq = lsq_ref[...] + 1e-9
    ld = jnp.sqrt(sq)
    env = jnp.clip(1.0 - ld * (1.0 / CUT), 0.0, 1.0) ** 5
    scale = env / ld
    for k in range(NR):
        fk = jnp.float32(jnp.pi * (k + 1) / CUT)
        o_ref[k] = jnp.sin(ld * fk) * scale


_tc_coef = pl.pallas_call(
    _tc_coef_body, out_shape=jax.ShapeDtypeStruct((NR, E_L // D, D), F32))


def _tc_rbf_body(c_ref, wr_ref, *o_refs):
    ct = c_ref[...]
    for b in range(NBLK):
        o_refs[b][...] = lax.dot_general(
            ct, wr_ref[b], (((0,), (0,)), ((), ())),
            preferred_element_type=F32)


_RBF_BM = 2048
_tc_rbf = pl.pallas_call(
    _tc_rbf_body,
    grid=(E_L // _RBF_BM,),
    in_specs=[
        pl.BlockSpec((NR, _RBF_BM), lambda i: (0, i)),
        pl.BlockSpec((NBLK, NR, D), lambda i: (0, 0, 0)),
    ],
    out_specs=[pl.BlockSpec((_RBF_BM, D), lambda i: (i, 0))
               for _ in range(NBLK)],
    out_shape=[jax.ShapeDtypeStruct((E_L, D), F32) for _ in range(NBLK)],
)


def _tc_pnode_body(part_ref, h_ref, wmsg_ref, wself_ref, o_ref):
    m = part_ref[0] + part_ref[1]
    x = (jnp.dot(m, wmsg_ref[...], preferred_element_type=F32)
         + jnp.dot(h_ref[...], wself_ref[...], preferred_element_type=F32))
    o_ref[...] = _leaky(x)


_PN_BM = 1000
_tc_pnode = pl.pallas_call(
    _tc_pnode_body,
    grid=(N_P // _PN_BM,),
    in_specs=[
        pl.BlockSpec((NC, _PN_BM, D), lambda i: (0, i, 0)),
        pl.BlockSpec((_PN_BM, D), lambda i: (i, 0)),
        pl.BlockSpec((D, D), lambda i: (0, 0)),
        pl.BlockSpec((D, D), lambda i: (0, 0)),
    ],
    out_specs=pl.BlockSpec((_PN_BM, D), lambda i: (i, 0)),
    out_shape=jax.ShapeDtypeStruct((N_P, D), F32),
)


def _tc_ab_body(h_ref, ws_ref, wd_ref, a_ref, b_ref):
    hl = h_ref[...]
    a_ref[...] = jnp.dot(hl, ws_ref[...], preferred_element_type=F32)
    b_ref[...] = jnp.dot(hl, wd_ref[...], preferred_element_type=F32)


_AB_BM = 1024
_tc_ab = pl.pallas_call(
    _tc_ab_body,
    grid=(N_L // _AB_BM,),
    in_specs=[
        pl.BlockSpec((_AB_BM, D), lambda i: (i, 0)),
        pl.BlockSpec((D, D), lambda i: (0, 0)),
        pl.BlockSpec((D, D), lambda i: (0, 0)),
    ],
    out_specs=[
        pl.BlockSpec((_AB_BM, D), lambda i: (i, 0)),
        pl.BlockSpec((_AB_BM, D), lambda i: (i, 0)),
    ],
    out_shape=[
        jax.ShapeDtypeStruct((N_L, D), F32),
        jax.ShapeDtypeStruct((N_L, D), F32),
    ],
)


def _tc_lupd_body(part_ref, hl_ref, wu_ref, o_ref):
    g = part_ref[0] + part_ref[1]
    x = jnp.dot(g, wu_ref[...], preferred_element_type=F32)
    o_ref[...] = _leaky(x) + hl_ref[...]


_tc_lupd = pl.pallas_call(
    _tc_lupd_body,
    grid=(N_L // _AB_BM,),
    in_specs=[
        pl.BlockSpec((NC, _AB_BM, D), lambda i: (0, i, 0)),
        pl.BlockSpec((_AB_BM, D), lambda i: (i, 0)),
        pl.BlockSpec((D, D), lambda i: (0, 0)),
    ],
    out_specs=pl.BlockSpec((_AB_BM, D), lambda i: (i, 0)),
    out_shape=jax.ShapeDtypeStruct((N_L, D), F32),
)


def _tc_lupd_ab_body(part_ref, hl_ref, wu_ref, ws_ref, wd_ref,
                     o_ref, a_ref, b_ref):
    g = part_ref[0] + part_ref[1]
    x = jnp.dot(g, wu_ref[...], preferred_element_type=F32)
    hl2 = _leaky(x) + hl_ref[...]
    o_ref[...] = hl2
    a_ref[...] = jnp.dot(hl2, ws_ref[...], preferred_element_type=F32)
    b_ref[...] = jnp.dot(hl2, wd_ref[...], preferred_element_type=F32)


_tc_lupd_ab = pl.pallas_call(
    _tc_lupd_ab_body,
    grid=(N_L // _AB_BM,),
    in_specs=[
        pl.BlockSpec((NC, _AB_BM, D), lambda i: (0, i, 0)),
        pl.BlockSpec((_AB_BM, D), lambda i: (i, 0)),
        pl.BlockSpec((D, D), lambda i: (0, 0)),
        pl.BlockSpec((D, D), lambda i: (0, 0)),
        pl.BlockSpec((D, D), lambda i: (0, 0)),
    ],
    out_specs=[pl.BlockSpec((_AB_BM, D), lambda i: (i, 0))] * 3,
    out_shape=[jax.ShapeDtypeStruct((N_L, D), F32)] * 3,
)


def _make_tc_pool(n, bm):
    def body(x_ref, bat_ref, o_ref, acc, cnt):
        i = pl.program_id(0)

        @pl.when(i == 0)
        def _():
            acc[...] = jnp.zeros_like(acc)
            cnt[...] = jnp.zeros_like(cnt)

        oh = (bat_ref[...] == lax.broadcasted_iota(I32, (1, BATCH), 1)
              ).astype(F32)
        x = x_ref[...]
        acc[...] += lax.dot_general(oh, x, (((0,), (0,)), ((), ())),
                                    preferred_element_type=F32)
        cnt[...] += lax.dot_general(oh, jnp.ones_like(x),
                                    (((0,), (0,)), ((), ())),
                                    preferred_element_type=F32)

        @pl.when(i == pl.num_programs(0) - 1)
        def _():
            o_ref[...] = acc[...] / jnp.maximum(cnt[...], 1.0)

    return pl.pallas_call(
        body,
        grid=(n // bm,),
        in_specs=[
            pl.BlockSpec((bm, D), lambda i: (i, 0)),
            pl.BlockSpec((bm, 1), lambda i: (i, 0)),
        ],
        out_specs=pl.BlockSpec((BATCH, D), lambda i: (0, 0)),
        out_shape=jax.ShapeDtypeStruct((BATCH, D), F32),
        scratch_shapes=[
            pltpu.VMEM((BATCH, D), F32),
            pltpu.VMEM((BATCH, D), F32),
        ],
    )


_tc_pool_l = _make_tc_pool(N_L, 1024)


def _tc_pnode_pool_body(part_ref, h_ref, wmsg_ref, wself_ref, bat_ref,
                        o_ref, acc, cnt):
    i = pl.program_id(0)

    @pl.when(i == 0)
    def _():
        acc[...] = jnp.zeros_like(acc)
        cnt[...] = jnp.zeros_like(cnt)

    m = part_ref[0] + part_ref[1]
    x = (jnp.dot(m, wmsg_ref[...], preferred_element_type=F32)
         + jnp.dot(h_ref[...], wself_ref[...], preferred_element_type=F32))
    y = _leaky(x)
    oh = (bat_ref[...] == lax.broadcasted_iota(I32, (1, BATCH), 1)
          ).astype(F32)
    acc[...] += lax.dot_general(oh, y, (((0,), (0,)), ((), ())),
                                preferred_element_type=F32)
    cnt[...] += lax.dot_general(oh, jnp.ones_like(y),
                                (((0,), (0,)), ((), ())),
                                preferred_element_type=F32)

    @pl.when(i == pl.num_programs(0) - 1)
    def _():
        o_ref[...] = acc[...] / jnp.maximum(cnt[...], 1.0)


_tc_pnode_pool = pl.pallas_call(
    _tc_pnode_pool_body,
    grid=(N_P // _PN_BM,),
    in_specs=[
        pl.BlockSpec((NC, _PN_BM, D), lambda i: (0, i, 0)),
        pl.BlockSpec((_PN_BM, D), lambda i: (i, 0)),
        pl.BlockSpec((D, D), lambda i: (0, 0)),
        pl.BlockSpec((D, D), lambda i: (0, 0)),
        pl.BlockSpec((_PN_BM, 1), lambda i: (i, 0)),
    ],
    out_specs=pl.BlockSpec((BATCH, D), lambda i: (0, 0)),
    out_shape=jax.ShapeDtypeStruct((BATCH, D), F32),
    scratch_shapes=[
        pltpu.VMEM((BATCH, D), F32),
        pltpu.VMEM((BATCH, D), F32),
    ],
)


def _tc_lupd_pool_body(part_ref, hl_ref, wu_ref, bat_ref, o_ref, acc, cnt):
    i = pl.program_id(0)

    @pl.when(i == 0)
    def _():
        acc[...] = jnp.zeros_like(acc)
        cnt[...] = jnp.zeros_like(cnt)

    g = part_ref[0] + part_ref[1]
    x = jnp.dot(g, wu_ref[...], preferred_element_type=F32)
    hl2 = _leaky(x) + hl_ref[...]
    oh = (bat_ref[...] == lax.broadcasted_iota(I32, (1, BATCH), 1)
          ).astype(F32)
    acc[...] += lax.dot_general(oh, hl2, (((0,), (0,)), ((), ())),
                                preferred_element_type=F32)
    cnt[...] += lax.dot_general(oh, jnp.ones_like(hl2),
                                (((0,), (0,)), ((), ())),
                                preferred_element_type=F32)

    @pl.when(i == pl.num_programs(0) - 1)
    def _():
        o_ref[...] = acc[...] / jnp.maximum(cnt[...], 1.0)


_tc_lupd_pool = pl.pallas_call(
    _tc_lupd_pool_body,
    grid=(N_L // _AB_BM,),
    in_specs=[
        pl.BlockSpec((NC, _AB_BM, D), lambda i: (0, i, 0)),
        pl.BlockSpec((_AB_BM, D), lambda i: (i, 0)),
        pl.BlockSpec((D, D), lambda i: (0, 0)),
        pl.BlockSpec((_AB_BM, 1), lambda i: (i, 0)),
    ],
    out_specs=pl.BlockSpec((BATCH, D), lambda i: (0, 0)),
    out_shape=jax.ShapeDtypeStruct((BATCH, D), F32),
    scratch_shapes=[
        pltpu.VMEM((BATCH, D), F32),
        pltpu.VMEM((BATCH, D), F32),
    ],
)


def _tc_fuse_body(pm, lm, prob, wph, bph, wlh, blh, w1p, w1l, w1r, b1,
                  w2, b2, w3, b3, o_ref):
    dot = functools.partial(jnp.dot, preferred_element_type=F32)
    pe = _leaky(dot(pm[...], wph[...]) + bph[...])
    le = _leaky(dot(lm[...], wlh[...]) + blh[...])
    x = _leaky(dot(pe, w1p[...]) + dot(le, w1l[...])
               + prob[...] * w1r[...] + b1[...])
    x = _leaky(dot(x, w2[...]) + b2[...])
    o_ref[...] = jax.nn.sigmoid(dot(x, w3[...]) + b3[...])


_tc_fuse = pl.pallas_call(
    _tc_fuse_body, out_shape=jax.ShapeDtypeStruct((BATCH, 1), F32))


def kernel(protein_x, protein_pos, protein_edge_index, protein_batch,
           ligand_z, ligand_pos, ligand_edge_index, ligand_batch,
           pocket_probability, Wp_msg, Wp_self, Wp_head, bp_head, emb_table,
           W_msg_b, W_upd_b, Wl_head, bl_head, W1, b1, W2, b2, W3, b3):
    psrc = protein_edge_index[0]
    pdst = protein_edge_index[1]
    lsrc = ligand_edge_index[0]
    ldst = ligand_edge_index[1]
    pdst4 = pdst.reshape(NT, NSUP, SCH, PK)
    ldst3 = ldst.reshape(NT, LCH, LK)

    pw, lsq, hl0 = _sc_geom(
        protein_pos[:, 0], protein_pos[:, 1], protein_pos[:, 2],
        psrc, pdst,
        ligand_pos[:, 0], ligand_pos[:, 1], ligand_pos[:, 2],
        lsrc, ldst, ligand_z, emb_table)

    coef = _tc_coef(lsq.reshape(E_L // D, D))
    redge = _tc_rbf(coef.reshape(NR, E_L), W_msg_b[:, 2 * D:, :])

    h = protein_x
    part = _sc_pmsg(h, pw, psrc, pdst4)
    h = _tc_pnode(part, h, Wp_msg[0], Wp_self[0])
    part = _sc_pmsg(h, pw, psrc, pdst4)
    pm = _tc_pnode_pool(part, h, Wp_msg[1], Wp_self[1],
                        protein_batch.reshape(N_P, 1))

    hl = hl0
    a, b = _tc_ab(hl, W_msg_b[0, :D, :], W_msg_b[0, D:2 * D, :])
    for blk in range(NBLK):
        part = _sc_lmsg(a, b, redge[blk], lsrc, ldst3)
        if blk < NBLK - 1:
            hl, a, b = _tc_lupd_ab(
                part, hl, W_upd_b[blk],
                W_msg_b[blk + 1, :D, :], W_msg_b[blk + 1, D:2 * D, :])
        else:
            lm = _tc_lupd_pool(part, hl, W_upd_b[blk],
                               ligand_batch.reshape(N_L, 1))

    return _tc_fuse(
        pm, lm, pocket_probability.reshape(BATCH, 1),
        Wp_head, bp_head.reshape(1, D), Wl_head, bl_head.reshape(1, D),
        W1[:D], W1[D:2 * D], W1[2 * D:].reshape(1, 64), b1.reshape(1, 64),
        W2, b2.reshape(1, 16), W3, b3.reshape(1, 1))


# speedup vs baseline: 1.1463x; 1.0010x over previous
import functools

import jax
import jax.numpy as jnp
from jax import lax
from jax.experimental import pallas as pl
from jax.experimental.pallas import tpu as pltpu
from jax.experimental.pallas import tpu_sc as plsc

F32 = jnp.float32
I32 = jnp.int32

NC = 2
NS = 16
NT = NC * NS

N_P = 10000
E_P = 320000
N_L = 4096
E_L = 65536
BATCH = 64
D = 128
NR = 6
CUT = 5.0
NBLK = 4

PE_T = E_P // NT
LE_T = E_L // NT
ZV_T = N_L // NT
PK = 80
PCH = PE_T // PK
LK = 64
LCH = LE_T // LK
N_PACC = 10240
PROWS_T = N_PACC // NS
LROWS_T = N_L // NS
ZR = 16
SB = 2000
NSUP = PE_T // SB
SCH = SB // PK
NV = D // 16

_sc_mesh = plsc.VectorSubcoreMesh(core_axis_name="c", subcore_axis_name="s")


def _leaky(x):
    return jnp.maximum(x, 0.01 * x)


@functools.partial(
    pl.kernel,
    out_type=(
        jax.ShapeDtypeStruct((E_P,), F32),
        jax.ShapeDtypeStruct((E_L,), F32),
        jax.ShapeDtypeStruct((N_L, D), F32),
    ),
    mesh=_sc_mesh,
    compiler_params=pltpu.CompilerParams(needs_layout_passes=False),
    scratch_types=[
        pltpu.VMEM((N_P,), F32),
        pltpu.VMEM((N_P,), F32),
        pltpu.VMEM((N_P,), F32),
        pltpu.VMEM((N_L,), F32),
        pltpu.VMEM((N_L,), F32),
        pltpu.VMEM((N_L,), F32),
        pltpu.VMEM((PE_T,), I32),
        pltpu.VMEM((PE_T,), I32),
        pltpu.VMEM((LE_T,), I32),
        pltpu.VMEM((LE_T,), I32),
        pltpu.VMEM((ZV_T,), I32),
        pltpu.VMEM((ZV_T, D), F32),
        pltpu.VMEM((PE_T,), F32),
        pltpu.VMEM((LE_T,), F32),
        pltpu.SemaphoreType.DMA,
    ],
)
def _sc_geom(ppx, ppy, ppz, psrc, pdst, lpx, lpy, lpz, lsrc, ldst, zidx, emb,
             psq_out, lsq_out, hl0_out,
             vpx, vpy, vpz, vlx, vly, vlz, vps, vpd, vls, vld, vz, vrows,
             vpsq, vlsq, sem):
    cid = lax.axis_index("c")
    sid = lax.axis_index("s")
    wid = cid * NS + sid

    pltpu.sync_copy(ppx, vpx)
    pltpu.sync_copy(ppy, vpy)
    pltpu.sync_copy(ppz, vpz)
    pltpu.sync_copy(lpx, vlx)
    pltpu.sync_copy(lpy, vly)
    pltpu.sync_copy(lpz, vlz)
    pltpu.sync_copy(psrc.at[pl.ds(wid * PE_T, PE_T)], vps)
    pltpu.sync_copy(pdst.at[pl.ds(wid * PE_T, PE_T)], vpd)
    pltpu.sync_copy(lsrc.at[pl.ds(wid * LE_T, LE_T)], vls)
    pltpu.sync_copy(ldst.at[pl.ds(wid * LE_T, LE_T)], vld)

    @pl.loop(0, PE_T // 16)
    def _(i):
        si = vps[pl.ds(i * 16, 16)]
        di = vpd[pl.ds(i * 16, 16)]
        dx = plsc.load_gather(vpx, [di]) - plsc.load_gather(vpx, [si])
        dy = plsc.load_gather(vpy, [di]) - plsc.load_gather(vpy, [si])
        dz = plsc.load_gather(vpz, [di]) - plsc.load_gather(vpz, [si])
        sq = dx * dx + dy * dy + dz * dz + 1e-9
        y = plsc.bitcast(0x5F3759DF - (plsc.bitcast(sq, I32) >> 1), F32)
        y = y * (1.5 - 0.5 * sq * y * y)
        y = y * (1.5 - 0.5 * sq * y * y)
        vpsq[pl.ds(i * 16, 16)] = jnp.exp(-(sq * y))

    @pl.loop(0, LE_T // 16)
    def _(i):
        si = vls[pl.ds(i * 16, 16)]
        di = vld[pl.ds(i * 16, 16)]
        dx = plsc.load_gather(vlx, [di]) - plsc.load_gather(vlx, [si])
        dy = plsc.load_gather(vly, [di]) - plsc.load_gather(vly, [si])
        dz = plsc.load_gather(vlz, [di]) - plsc.load_gather(vlz, [si])
        vlsq[pl.ds(i * 16, 16)] = dx * dx + dy * dy + dz * dz

    pltpu.sync_copy(vpsq, psq_out.at[pl.ds(wid * PE_T, PE_T)])
    pltpu.sync_copy(vlsq, lsq_out.at[pl.ds(wid * LE_T, LE_T)])

    pltpu.sync_copy(zidx.at[pl.ds(wid * ZV_T, ZV_T)], vz)
    pltpu.async_copy(emb.at[vz], vrows, sem).wait()
    pltpu.sync_copy(vrows, hl0_out.at[pl.ds(wid * ZV_T, ZV_T)])


@functools.partial(
    pl.kernel,
    out_type=jax.ShapeDtypeStruct((NC, N_PACC, D), F32),
    mesh=_sc_mesh,
    compiler_params=pltpu.CompilerParams(needs_layout_passes=False),
    scratch_types=[
        pltpu.VMEM_SHARED((N_PACC, D), F32),
        pltpu.VMEM((SB,), I32),
        pltpu.VMEM((SCH, PK), I32),
        pltpu.VMEM((SB,), F32),
        pltpu.VMEM((3, PK, D), F32),
        pltpu.VMEM((ZR, D), F32),
        pltpu.SemaphoreType.DMA,
        pltpu.SemaphoreType.DMA,
        pltpu.SemaphoreType.DMA,
        pltpu.SemaphoreType.DMA,
        pltpu.SemaphoreType.DMA,
        pltpu.SemaphoreType.DMA,
    ],
)
def _sc_pmsg(h, pw, srcidx, dstidx4, out, acc, vsrc, vdst, vw,
             gbuf3, zbuf, g0, g1, g2, s0, s1, s2):
    cid = lax.axis_index("c")
    sid = lax.axis_index("s")
    wid = cid * NS + sid

    @pl.loop(0, ZR)
    def _(r):
        for v in range(NV):
            zbuf[r, pl.ds(v * 16, 16)] = jnp.zeros((16,), F32)

    row0 = sid * PROWS_T

    @pl.loop(0, PROWS_T // ZR)
    def _(t):
        pltpu.sync_copy(zbuf, acc.at[pl.ds(row0 + t * ZR, ZR)])

    plsc.subcore_barrier()

    gsems = (g0, g1, g2)
    ssems = (s0, s1, s2)

    def _issue(j, b):
        pltpu.async_copy(h.at[vsrc.at[pl.ds(j * PK, PK)]], gbuf3.at[b],
                         gsems[b])

    def _drain(b):
        pltpu.make_async_copy(gbuf3.at[b], acc.at[vdst.at[0]],
                              ssems[b]).wait()

    def _process(j, b):
        buf = gbuf3.at[b]
        pltpu.make_async_copy(h.at[vsrc.at[pl.ds(0, PK)]], buf,
                              gsems[b]).wait()

        @pl.loop(0, PK // 16)
        def _(e16):
            wv = vw[pl.ds(j * PK + e16 * 16, 16)]
            for u in range(16):
                w = wv[u]
                r = e16 * 16 + u
                for v in range(NV):
                    buf[r, pl.ds(v * 16, 16)] = buf[r, pl.ds(v * 16, 16)] * w

        pltpu.async_copy(buf, acc.at[vdst.at[j]], ssems[b], add=True)

    @pl.loop(0, NSUP)
    def _(sc):
        base = wid * PE_T + sc * SB
        pltpu.sync_copy(srcidx.at[pl.ds(base, SB)], vsrc)
        pltpu.sync_copy(pw.at[pl.ds(base, SB)], vw)
        pltpu.sync_copy(dstidx4.at[wid, sc], vdst)

        _issue(0, 0)
        _issue(1, 1)
        for j in range(SCH):
            _process(j, j % 3)
            if j + 2 < SCH:
                if j >= 1:
                    _drain((j + 2) % 3)
                _issue(j + 2, (j + 2) % 3)
        for j in range(SCH - 3, SCH):
            _drain(j % 3)

    plsc.subcore_barrier()
    pltpu.sync_copy(acc.at[pl.ds(sid * PROWS_T, PROWS_T)],
                    out.at[cid, pl.ds(sid * PROWS_T, PROWS_T)])


@functools.partial(
    pl.kernel,
    out_type=jax.ShapeDtypeStruct((NC, N_L, D), F32),
    mesh=_sc_mesh,
    compiler_params=pltpu.CompilerParams(needs_layout_passes=False),
    scratch_types=[
        pltpu.VMEM_SHARED((N_L, D), F32),
        pltpu.VMEM((LE_T,), I32),
        pltpu.VMEM((LCH, LK), I32),
        pltpu.VMEM((3, LK, D), F32),
        pltpu.VMEM((3, LK, D), F32),
        pltpu.VMEM((3, LK, D), F32),
        pltpu.VMEM((ZR, D), F32),
        pltpu.SemaphoreType.DMA,
        pltpu.SemaphoreType.DMA,
        pltpu.SemaphoreType.DMA,
        pltpu.SemaphoreType.DMA,
        pltpu.SemaphoreType.DMA,
        pltpu.SemaphoreType.DMA,
    ],
)
def _sc_lmsg(a_nodes, b_nodes, redge, srcidx, dstidx3, out,
             acc, vsrc, vdst, ga3, gb3, rbuf3, zbuf, g0, g1, g2, s0, s1, s2):
    cid = lax.axis_index("c")
    sid = lax.axis_index("s")
    wid = cid * NS + sid

    @pl.loop(0, ZR)
    def _(r):
        for v in range(NV):
            zbuf[r, pl.ds(v * 16, 16)] = jnp.zeros((16,), F32)

    row0 = sid * LROWS_T

    @pl.loop(0, LROWS_T // ZR)
    def _(t):
        pltpu.sync_copy(zbuf, acc.at[pl.ds(row0 + t * ZR, ZR)])

    plsc.subcore_barrier()

    pltpu.sync_copy(srcidx.at[pl.ds(wid * LE_T, LE_T)], vsrc)
    pltpu.sync_copy(dstidx3.at[wid], vdst)

    gsems = (g0, g1, g2)
    ssems = (s0, s1, s2)

    def _issue(j, b):
        sem = gsems[b]
        pltpu.async_copy(
            a_nodes.at[vsrc.at[pl.ds(j * LK, LK)]], ga3.at[b], sem)
        pltpu.async_copy(b_nodes.at[vdst.at[j]], gb3.at[b], sem)
        pltpu.async_copy(
            redge.at[pl.ds(wid * LE_T + j * LK, LK)], rbuf3.at[b], sem)

    def _drain(b):
        pltpu.make_async_copy(ga3.at[b], acc.at[vdst.at[0]], ssems[b]).wait()

    def _process(j, b):
        for _ in range(3):
            pltpu.make_async_copy(
                a_nodes.at[vsrc.at[pl.ds(0, LK)]], ga3.at[b],
                gsems[b]).wait()
        ga = ga3.at[b]
        gb = gb3.at[b]
        rbuf = rbuf3.at[b]

        @pl.loop(0, LK)
        def _(e):
            for v in range(NV):
                x = (ga[e, pl.ds(v * 16, 16)] + gb[e, pl.ds(v * 16, 16)]
                     + rbuf[e, pl.ds(v * 16, 16)])
                ga[e, pl.ds(v * 16, 16)] = jnp.maximum(x, 0.01 * x)

        pltpu.async_copy(ga, acc.at[vdst.at[j]], ssems[b], add=True)

    _issue(0, 0)
    _issue(1, 1)
    for j in range(LCH):
        _process(j, j % 3)
        if j + 2 < LCH:
            if j >= 1:
                _drain((j + 2) % 3)
            _issue(j + 2, (j + 2) % 3)
    for j in range(LCH - 3, LCH):
        _drain(j % 3)

    plsc.subcore_barrier()
    pltpu.sync_copy(acc.at[pl.ds(sid * LROWS_T, LROWS_T)],
                    out.at[cid, pl.ds(sid * LROWS_T, LROWS_T)])


def _tc_coef_body(lsq_ref, o_ref):
    sq = lsq_ref[...] + 1e-9
    ld = jnp.sqrt(sq)
    env = jnp.clip(1.0 - ld * (1.0 / CUT), 0.0, 1.0) ** 5
    scale = env / ld
    for k in range(NR):
        fk = jnp.float32(jnp.pi * (k + 1) / CUT)
        o_ref[k] = jnp.sin(ld * fk) * scale


_tc_coef = pl.pallas_call(
    _tc_coef_body, out_shape=jax.ShapeDtypeStruct((NR, E_L // D, D), F32))


def _tc_rbf_body(c_ref, wr_ref, *o_refs):
    ct = c_ref[...]
    for b in range(NBLK):
        o_refs[b][...] = lax.dot_general(
            ct, wr_ref[b], (((0,), (0,)), ((), ())),
            preferred_element_type=F32)


_RBF_BM = 2048
_tc_rbf = pl.pallas_call(
    _tc_rbf_body,
    grid=(E_L // _RBF_BM,),
    in_specs=[
        pl.BlockSpec((NR, _RBF_BM), lambda i: (0, i)),
        pl.BlockSpec((NBLK, NR, D), lambda i: (0, 0, 0)),
    ],
    out_specs=[pl.BlockSpec((_RBF_BM, D), lambda i: (i, 0))
               for _ in range(NBLK)],
    out_shape=[jax.ShapeDtypeStruct((E_L, D), F32) for _ in range(NBLK)],
)


def _tc_pnode_body(part_ref, h_ref, wmsg_ref, wself_ref, o_ref):
    m = part_ref[0] + part_ref[1]
    x = (jnp.dot(m, wmsg_ref[...], preferred_element_type=F32)
         + jnp.dot(h_ref[...], wself_ref[...], preferred_element_type=F32))
    o_ref[...] = _leaky(x)


_PN_BM = 1000
_tc_pnode = pl.pallas_call(
    _tc_pnode_body,
    grid=(N_P // _PN_BM,),
    in_specs=[
        pl.BlockSpec((NC, _PN_BM, D), lambda i: (0, i, 0)),
        pl.BlockSpec((_PN_BM, D), lambda i: (i, 0)),
        pl.BlockSpec((D, D), lambda i: (0, 0)),
        pl.BlockSpec((D, D), lambda i: (0, 0)),
    ],
    out_specs=pl.BlockSpec((_PN_BM, D), lambda i: (i, 0)),
    out_shape=jax.ShapeDtypeStruct((N_P, D), F32),
)


def _tc_ab_body(h_ref, ws_ref, wd_ref, a_ref, b_ref):
    hl = h_ref[...]
    a_ref[...] = jnp.dot(hl, ws_ref[...], preferred_element_type=F32)
    b_ref[...] = jnp.dot(hl, wd_ref[...], preferred_element_type=F32)


_AB_BM = 1024
_tc_ab = pl.pallas_call(
    _tc_ab_body,
    grid=(N_L // _AB_BM,),
    in_specs=[
        pl.BlockSpec((_AB_BM, D), lambda i: (i, 0)),
        pl.BlockSpec((D, D), lambda i: (0, 0)),
        pl.BlockSpec((D, D), lambda i: (0, 0)),
    ],
    out_specs=[
        pl.BlockSpec((_AB_BM, D), lambda i: (i, 0)),
        pl.BlockSpec((_AB_BM, D), lambda i: (i, 0)),
    ],
    out_shape=[
        jax.ShapeDtypeStruct((N_L, D), F32),
        jax.ShapeDtypeStruct((N_L, D), F32),
    ],
)


def _tc_lupd_ab_body(part_ref, hl_ref, wu_ref, ws_ref, wd_ref,
                     o_ref, a_ref, b_ref):
    g = part_ref[0] + part_ref[1]
    x = jnp.dot(g, wu_ref[...], preferred_element_type=F32)
    hl2 = _leaky(x) + hl_ref[...]
    o_ref[...] = hl2
    a_ref[...] = jnp.dot(hl2, ws_ref[...], preferred_element_type=F32)
    b_ref[...] = jnp.dot(hl2, wd_ref[...], preferred_element_type=F32)


_tc_lupd_ab = pl.pallas_call(
    _tc_lupd_ab_body,
    grid=(N_L // _AB_BM,),
    in_specs=[
        pl.BlockSpec((NC, _AB_BM, D), lambda i: (0, i, 0)),
        pl.BlockSpec((_AB_BM, D), lambda i: (i, 0)),
        pl.BlockSpec((D, D), lambda i: (0, 0)),
        pl.BlockSpec((D, D), lambda i: (0, 0)),
        pl.BlockSpec((D, D), lambda i: (0, 0)),
    ],
    out_specs=[pl.BlockSpec((_AB_BM, D), lambda i: (i, 0))] * 3,
    out_shape=[jax.ShapeDtypeStruct((N_L, D), F32)] * 3,
)


def _tc_pnode_pool_body(part_ref, h_ref, wmsg_ref, wself_ref, bat_ref,
                        o_ref, acc, cnt):
    i = pl.program_id(0)

    @pl.when(i == 0)
    def _():
        acc[...] = jnp.zeros_like(acc)
        cnt[...] = jnp.zeros_like(cnt)

    m = part_ref[0] + part_ref[1]
    x = (jnp.dot(m, wmsg_ref[...], preferred_element_type=F32)
         + jnp.dot(h_ref[...], wself_ref[...], preferred_element_type=F32))
    y = _leaky(x)
    oh = (bat_ref[...] == lax.broadcasted_iota(I32, (1, BATCH), 1)
          ).astype(F32)
    acc[...] += lax.dot_general(oh, y, (((0,), (0,)), ((), ())),
                                preferred_element_type=F32)
    cnt[...] += lax.dot_general(oh, jnp.ones_like(y),
                                (((0,), (0,)), ((), ())),
                                preferred_element_type=F32)

    @pl.when(i == pl.num_programs(0) - 1)
    def _():
        o_ref[...] = acc[...] / jnp.maximum(cnt[...], 1.0)


_tc_pnode_pool = pl.pallas_call(
    _tc_pnode_pool_body,
    grid=(N_P // _PN_BM,),
    in_specs=[
        pl.BlockSpec((NC, _PN_BM, D), lambda i: (0, i, 0)),
        pl.BlockSpec((_PN_BM, D), lambda i: (i, 0)),
        pl.BlockSpec((D, D), lambda i: (0, 0)),
        pl.BlockSpec((D, D), lambda i: (0, 0)),
        pl.BlockSpec((_PN_BM, 1), lambda i: (i, 0)),
    ],
    out_specs=pl.BlockSpec((BATCH, D), lambda i: (0, 0)),
    out_shape=jax.ShapeDtypeStruct((BATCH, D), F32),
    scratch_shapes=[
        pltpu.VMEM((BATCH, D), F32),
        pltpu.VMEM((BATCH, D), F32),
    ],
)


def _tc_lupd_pool_body(part_ref, hl_ref, wu_ref, bat_ref, o_ref, acc, cnt):
    i = pl.program_id(0)

    @pl.when(i == 0)
    def _():
        acc[...] = jnp.zeros_like(acc)
        cnt[...] = jnp.zeros_like(cnt)

    g = part_ref[0] + part_ref[1]
    x = jnp.dot(g, wu_ref[...], preferred_element_type=F32)
    hl2 = _leaky(x) + hl_ref[...]
    oh = (bat_ref[...] == lax.broadcasted_iota(I32, (1, BATCH), 1)
          ).astype(F32)
    acc[...] += lax.dot_general(oh, hl2, (((0,), (0,)), ((), ())),
                                preferred_element_type=F32)
    cnt[...] += lax.dot_general(oh, jnp.ones_like(hl2),
                                (((0,), (0,)), ((), ())),
                                preferred_element_type=F32)

    @pl.when(i == pl.num_programs(0) - 1)
    def _():
        o_ref[...] = acc[...] / jnp.maximum(cnt[...], 1.0)


_tc_lupd_pool = pl.pallas_call(
    _tc_lupd_pool_body,
    grid=(N_L // _AB_BM,),
    in_specs=[
        pl.BlockSpec((NC, _AB_BM, D), lambda i: (0, i, 0)),
        pl.BlockSpec((_AB_BM, D), lambda i: (i, 0)),
        pl.BlockSpec((D, D), lambda i: (0, 0)),
        pl.BlockSpec((_AB_BM, 1), lambda i: (i, 0)),
    ],
    out_specs=pl.BlockSpec((BATCH, D), lambda i: (0, 0)),
    out_shape=jax.ShapeDtypeStruct((BATCH, D), F32),
    scratch_shapes=[
        pltpu.VMEM((BATCH, D), F32),
        pltpu.VMEM((BATCH, D), F32),
    ],
)


def _tc_fuse_body(pm, lm, prob, wph, bph, wlh, blh, w1p, w1l, w1r, b1,
                  w2, b2, w3, b3, o_ref):
    dot = functools.partial(jnp.dot, preferred_element_type=F32)
    pe = _leaky(dot(pm[...], wph[...]) + bph[...])
    le = _leaky(dot(lm[...], wlh[...]) + blh[...])
    x = _leaky(dot(pe, w1p[...]) + dot(le, w1l[...])
               + prob[...] * w1r[...] + b1[...])
    x = _leaky(dot(x, w2[...]) + b2[...])
    o_ref[...] = jax.nn.sigmoid(dot(x, w3[...]) + b3[...])


_tc_fuse = pl.pallas_call(
    _tc_fuse_body, out_shape=jax.ShapeDtypeStruct((BATCH, 1), F32))


def kernel(protein_x, protein_pos, protein_edge_index, protein_batch,
           ligand_z, ligand_pos, ligand_edge_index, ligand_batch,
           pocket_probability, Wp_msg, Wp_self, Wp_head, bp_head, emb_table,
           W_msg_b, W_upd_b, Wl_head, bl_head, W1, b1, W2, b2, W3, b3):
    psrc = protein_edge_index[0]
    pdst = protein_edge_index[1]
    lsrc = ligand_edge_index[0]
    ldst = ligand_edge_index[1]
    pdst4 = pdst.reshape(NT, NSUP, SCH, PK)
    ldst3 = ldst.reshape(NT, LCH, LK)

    pw, lsq, hl0 = _sc_geom(
        protein_pos[:, 0], protein_pos[:, 1], protein_pos[:, 2],
        psrc, pdst,
        ligand_pos[:, 0], ligand_pos[:, 1], ligand_pos[:, 2],
        lsrc, ldst, ligand_z, emb_table)

    coef = _tc_coef(lsq.reshape(E_L // D, D))
    redge = _tc_rbf(coef.reshape(NR, E_L), W_msg_b[:, 2 * D:, :])

    h = protein_x
    part = _sc_pmsg(h, pw, psrc, pdst4)
    h = _tc_pnode(part, h, Wp_msg[0], Wp_self[0])
    part = _sc_pmsg(h, pw, psrc, pdst4)
    pm = _tc_pnode_pool(part, h, Wp_msg[1], Wp_self[1],
                        protein_batch.reshape(N_P, 1))

    hl = hl0
    a, b = _tc_ab(hl, W_msg_b[0, :D, :], W_msg_b[0, D:2 * D, :])
    for blk in range(NBLK):
        part = _sc_lmsg(a, b, redge[blk], lsrc, ldst3)
        if blk < NBLK - 1:
            hl, a, b = _tc_lupd_ab(
                part, hl, W_upd_b[blk],
                W_msg_b[blk + 1, :D, :], W_msg_b[blk + 1, D:2 * D, :])
        else:
            lm = _tc_lupd_pool(part, hl, W_upd_b[blk],
                               ligand_batch.reshape(N_L, 1))

    return _tc_fuse(
        pm, lm, pocket_probability.reshape(BATCH, 1),
        Wp_head, bp_head.reshape(1, D), Wl_head, bl_head.reshape(1, D),
        W1[:D], W1[D:2 * D], W1[2 * D:].reshape(1, 64), b1.reshape(1, 64),
        W2, b2.reshape(1, 16), W3, b3.reshape(1, 1))
